# Initial kernel scaffold; baseline (speedup 1.0000x reference)
import jax
import jax.numpy as jnp
from jax.experimental import pallas as pl


def kernel(input_ids, table):
    raise NotImplementedError("write your pallas kernel here")


# speedup vs baseline: 1.8327x; 1.8327x over previous
import jax
import jax.numpy as jnp
from jax import lax
from jax.experimental import pallas as pl
from jax.experimental.pallas import tpu as pltpu
from jax.experimental.pallas import tpu_sc as plsc

VOCAB = 1000000
EMBED_DIM = 64
BATCH = 16384
HIST = 50

NC, NS = 2, 16
NW = NC * NS
B_TOTAL = BATCH * HIST
PER_W = B_TOTAL // NW

IDX_W = 128
IDX_ROWS_W = PER_W // IDX_W
CHUNK = 512
IDX_ROWS_CHUNK = CHUNK // IDX_W
N_CHUNKS = PER_W // CHUNK


def _body(table_h, idx_h, out_h, idx_v, rows_v, sem):
    wid = lax.axis_index("s") * NC + lax.axis_index("c")
    idx_row0 = wid * IDX_ROWS_W
    out0 = wid * PER_W

    pltpu.sync_copy(idx_h.at[pl.ds(idx_row0, IDX_ROWS_W)], idx_v)

    @pl.loop(0, N_CHUNKS)
    def _(g):
        descs = []
        for j in range(IDX_ROWS_CHUNK):
            descs.append(pltpu.async_copy(
                table_h.at[idx_v.at[g * IDX_ROWS_CHUNK + j]],
                rows_v.at[pl.ds(j * IDX_W, IDX_W)],
                sem))
        for d in descs:
            d.wait()
        pltpu.sync_copy(rows_v, out_h.at[pl.ds(out0 + g * CHUNK, CHUNK)])


_gather = pl.kernel(
    _body,
    out_type=jax.ShapeDtypeStruct((B_TOTAL, EMBED_DIM), jnp.float32),
    mesh=plsc.VectorSubcoreMesh(core_axis_name="c", subcore_axis_name="s"),
    scratch_types=[
        pltpu.VMEM((IDX_ROWS_W, IDX_W), jnp.int32),
        pltpu.VMEM((CHUNK, EMBED_DIM), jnp.float32),
        pltpu.SemaphoreType.DMA,
    ],
    compiler_params=pltpu.CompilerParams(use_tc_tiling_on_sc=False),
)


def kernel(input_ids, table):
    idx = input_ids.astype(jnp.int32).reshape(B_TOTAL // IDX_W, IDX_W)
    out = _gather(table, idx)
    return out.reshape(BATCH, HIST, EMBED_DIM)


# speedup vs baseline: 1.8753x; 1.0233x over previous
import jax
import jax.numpy as jnp
from jax import lax
from jax.experimental import pallas as pl
from jax.experimental.pallas import tpu as pltpu
from jax.experimental.pallas import tpu_sc as plsc

VOCAB = 1000000
EMBED_DIM = 64
BATCH = 16384
HIST = 50

NC, NS = 2, 16
NW = NC * NS
B_TOTAL = BATCH * HIST
PER_W = B_TOTAL // NW

IDX_W = 128
IDX_ROWS_W = PER_W // IDX_W
CHUNK = 512
IDX_ROWS_CHUNK = CHUNK // IDX_W
N_CHUNKS = PER_W // CHUNK


def _body(table_h, idx_h, out_h, idx_v, buf0, buf1, sem0, sem1):
    wid = lax.axis_index("s") * NC + lax.axis_index("c")
    idx_row0 = wid * IDX_ROWS_W
    out0 = wid * PER_W

    pltpu.sync_copy(idx_h.at[pl.ds(idx_row0, IDX_ROWS_W)], idx_v)

    def fire(g, buf, sem):
        for j in range(IDX_ROWS_CHUNK):
            pltpu.async_copy(
                table_h.at[idx_v.at[g * IDX_ROWS_CHUNK + j]],
                buf.at[pl.ds(j * IDX_W, IDX_W)],
                sem)

    def drain(buf, sem):
        pltpu.make_async_copy(table_h.at[pl.ds(0, CHUNK)], buf, sem).wait()

    def write(g, buf):
        pltpu.sync_copy(buf, out_h.at[pl.ds(out0 + g * CHUNK, CHUNK)])

    @pl.loop(0, N_CHUNKS // 2)
    def _(i):
        g = 2 * i
        fire(g, buf0, sem0)

        @pl.when(g > 0)
        def _():
            drain(buf1, sem1)
            write(g - 1, buf1)

        fire(g + 1, buf1, sem1)
        drain(buf0, sem0)
        write(g, buf0)

    drain(buf1, sem1)
    write(N_CHUNKS - 1, buf1)


_gather = pl.kernel(
    _body,
    out_type=jax.ShapeDtypeStruct((B_TOTAL, EMBED_DIM), jnp.float32),
    mesh=plsc.VectorSubcoreMesh(core_axis_name="c", subcore_axis_name="s"),
    scratch_types=[
        pltpu.VMEM((IDX_ROWS_W, IDX_W), jnp.int32),
        pltpu.VMEM((CHUNK, EMBED_DIM), jnp.float32),
        pltpu.VMEM((CHUNK, EMBED_DIM), jnp.float32),
        pltpu.SemaphoreType.DMA,
        pltpu.SemaphoreType.DMA,
    ],
    compiler_params=pltpu.CompilerParams(use_tc_tiling_on_sc=False),
)


def kernel(input_ids, table):
    idx = input_ids.astype(jnp.int32).reshape(B_TOTAL // IDX_W, IDX_W)
    out = _gather(table, idx)
    return out.reshape(BATCH, HIST, EMBED_DIM)
